# baseline (speedup 1.0000x reference)
import functools

import jax
import jax.numpy as jnp
from jax import lax
from jax.experimental import pallas as pl
from jax.experimental.pallas import tpu as pltpu
from jax.experimental.pallas import tpu_sc as plsc

NB = 4


@functools.lru_cache(maxsize=None)
def _make_sc_gather(batch, seqlen, embed, vocab):
    info = plsc.get_sparse_core_info()
    nw = info.num_cores * info.num_subcores
    assert batch % nw == 0
    chunks = batch // nw
    assert chunks % NB == 0 and chunks >= NB
    mesh = plsc.VectorSubcoreMesh(core_axis_name="c", subcore_axis_name="s")

    @functools.partial(
        pl.kernel,
        mesh=mesh,
        compiler_params=pltpu.CompilerParams(use_tc_tiling_on_sc=False),
        out_type=jax.ShapeDtypeStruct((batch, seqlen, embed), jnp.float32),
        scratch_types=(
            [pltpu.VMEM((seqlen, embed), jnp.float32) for _ in range(NB)]
            + [pltpu.VMEM((seqlen,), jnp.int32) for _ in range(NB)]
            + [pltpu.SemaphoreType.DMA for _ in range(3 * NB)]
        ),
    )
    def k(x_hbm, tok_hbm, out_hbm, *scratch):
        rows = scratch[:NB]
        idxs = scratch[NB:2 * NB]
        isem = scratch[2 * NB:3 * NB]
        gsem = scratch[3 * NB:4 * NB]
        ssem = scratch[4 * NB:]

        wid = lax.axis_index("s") * info.num_cores + lax.axis_index("c")
        row0 = wid * chunks

        def x_slice(i):
            return x_hbm.at[pl.ds((row0 + i) * seqlen, seqlen)]

        def out_slice(i):
            return out_hbm.at[row0 + i]

        def fetch(i, b):
            @pl.when(i < chunks)
            def _():
                @pl.when(i >= NB)
                def _():
                    pltpu.make_async_copy(rows[b], out_slice(i - NB), ssem[b]).wait()

                pltpu.async_copy(x_slice(i), idxs[b], isem[b])

        def gather(i, b):
            @pl.when(jnp.logical_and(i >= 0, i < chunks))
            def _():
                pltpu.make_async_copy(x_slice(i), idxs[b], isem[b]).wait()
                pltpu.async_copy(tok_hbm.at[idxs[b]], rows[b], gsem[b])

        def store(i, b):
            @pl.when(jnp.logical_and(i >= 0, i < chunks))
            def _():
                pltpu.make_async_copy(tok_hbm.at[idxs[b]], rows[b], gsem[b]).wait()
                pltpu.async_copy(rows[b], out_slice(i), ssem[b])

        def visit_group(kk, _):
            for j in range(NB):
                v = NB * kk + j - 2
                fetch(v + 2, j)
                gather(v + 1, (j + 3) % NB)
                store(v, (j + 2) % NB)
            return 0

        lax.fori_loop(0, chunks // NB + 1, visit_group, 0)

        for b in range(NB):
            pltpu.make_async_copy(rows[b], out_slice(0), ssem[b]).wait()

    return k


def kernel(x, token_table, pos_table):
    batch, seqlen = x.shape
    vocab, embed = token_table.shape
    k = _make_sc_gather(batch, seqlen, embed, vocab)
    tok = k(x.reshape(-1).astype(jnp.int32), token_table)
    return tok + pos_table[None, :, :]
